# baseline (device time: 38366 ns/iter reference)
import numpy as np
import jax
import jax.numpy as jnp
from jax import lax
from jax.experimental import pallas as pl
from jax.experimental.pallas import tpu as pltpu

N_DEV = 4


def _rope_tables(Sq, Dh, n_heads, Bl):
    inv = 1.0 / (10000.0 ** (np.arange(0, Dh, 2) / Dh))
    pos = np.arange(Sq)[:, None] * inv[None, :]
    cos = np.repeat(np.cos(pos), 2, axis=-1).astype(np.float32)
    sin = np.repeat(np.sin(pos), 2, axis=-1).astype(np.float32)
    cos_t = np.tile(np.tile(cos, (1, n_heads)), (Bl, 1))
    sin_t = np.tile(np.tile(sin, (1, n_heads)), (Bl, 1))
    n = n_heads * Dh
    P = np.zeros((n, n), np.float32)
    for h in range(n_heads):
        o = h * Dh
        for k in range(Dh // 2):
            P[o + 2 * k + 1, o + 2 * k] = -1.0
            P[o + 2 * k, o + 2 * k + 1] = 1.0
    return cos_t, sin_t, P


def kernel(x, Wq, Wk, Wv, Wo):
    Bl, Sq, D = x.shape
    HD = Wq.shape[1]
    Dh = 64
    HW = HD // 2
    BS = Bl * Sq
    NH = N_DEV - 1

    wcat = jnp.concatenate([Wq, Wk, Wv, Wo.T], axis=0).astype(jnp.bfloat16)
    cos_np, sin_np, P_np = _rope_tables(Sq, Dh, HD // Dh, Bl)
    cos_t = jnp.asarray(cos_np)
    sin_t = jnp.asarray(sin_np)
    P_m = jnp.asarray(P_np).astype(jnp.bfloat16)

    def body(x_ref, wcat_ref, cos_ref, sin_ref, p_ref, out_ref,
             wcw_ref, wccw_ref, ctx_ref,
             cw_send, cw_recv, ccw_send, ccw_recv):
        me = lax.axis_index("i")
        left = lax.rem(me + N_DEV - 1, N_DEV)
        right = lax.rem(me + 1, N_DEV)

        barrier = pltpu.get_barrier_semaphore()
        for nbr in (left, right):
            pl.semaphore_signal(
                barrier, inc=1,
                device_id=(nbr,), device_id_type=pl.DeviceIdType.MESH,
            )
        pl.semaphore_wait(barrier, 2)

        CHUNK_ROWS = ((0, 2 * D), (2 * D, 3 * D), (3 * D, 4 * D))
        NC = len(CHUNK_ROWS)

        def make_chunk(u, c, ccw):
            w_ref = wccw_ref if ccw else wcw_ref
            lo, hi = CHUNK_ROWS[c]
            if u == 0:
                cols = (HW, HD) if ccw else (0, HW)
                src = wcat_ref.at[lo:hi, cols[0]:cols[1]]
            else:
                src = w_ref.at[u - 1, lo:hi, :]
            sems = (ccw_send, ccw_recv) if ccw else (cw_send, cw_recv)
            tgt = left if ccw else right
            r = pltpu.make_async_remote_copy(
                src_ref=src, dst_ref=w_ref.at[u, lo:hi, :],
                send_sem=sems[0].at[u, c], recv_sem=sems[1].at[u, c],
                device_id=(tgt,), device_id_type=pl.DeviceIdType.MESH,
            )
            r.start()
            return r

        rd = {}
        for c in range(NC):
            rd["cw", 0, c] = make_chunk(0, c, ccw=False)
            rd["ccw", 0, c] = make_chunk(0, c, ccw=True)

        x2 = x_ref[...].reshape(BS, D)
        cos2 = cos_ref[...]
        sin2 = sin_ref[...]
        pm = p_ref[...]

        def attn_weights(wqk):
            width = wqk.shape[1]
            cw_ = cos2[:, :width]
            sw_ = sin2[:, :width]
            pw_ = pm[:width, :width]
            q = jnp.dot(x2, wqk[0:D], preferred_element_type=jnp.float32)
            k = jnp.dot(x2, wqk[D:2 * D], preferred_element_type=jnp.float32)
            qp = jnp.dot(q.astype(jnp.bfloat16), pw_, preferred_element_type=jnp.float32)
            kp = jnp.dot(k.astype(jnp.bfloat16), pw_, preferred_element_type=jnp.float32)
            q = (q * cw_ + qp * sw_).astype(jnp.bfloat16)
            k = (k * cw_ + kp * sw_).astype(jnp.bfloat16)
            ws = {}
            for b in range(Bl):
                for h in range(width // Dh):
                    qh = q[b * Sq:(b + 1) * Sq, h * Dh:(h + 1) * Dh]
                    kh = k[b * Sq:(b + 1) * Sq, h * Dh:(h + 1) * Dh]
                    s = lax.dot_general(
                        qh, kh, (((1,), (1,)), ((), ())),
                        preferred_element_type=jnp.float32,
                    ) * 0.125
                    w = jnp.exp(s)
                    denom = jnp.sum(w, axis=-1, keepdims=True)
                    ws[b, h] = (w.astype(jnp.bfloat16), denom)
            return ws

        def attn_ctx(wv, ws):
            width = wv.shape[1]
            v = jnp.dot(
                x2, wv, preferred_element_type=jnp.float32
            ).astype(jnp.bfloat16)
            for b in range(Bl):
                for h in range(width // Dh):
                    vh = v[b * Sq:(b + 1) * Sq, h * Dh:(h + 1) * Dh]
                    w, denom = ws[b, h]
                    cx = jnp.dot(
                        w, vh, preferred_element_type=jnp.float32
                    ) / denom
                    ctx_ref[b * Sq:(b + 1) * Sq, h * Dh:(h + 1) * Dh] = (
                        cx.astype(jnp.bfloat16)
                    )

        def out_proj(woT):
            width = woT.shape[1]
            return lax.dot_general(
                ctx_ref[:, :width], woT, (((1,), (1,)), ((), ())),
                preferred_element_type=jnp.float32,
            )

        def compute_block(wblk):
            ws = attn_weights(wblk[0:2 * D])
            attn_ctx(wblk[2 * D:3 * D], ws)
            return out_proj(wblk[3 * D:4 * D])

        acc = compute_block(wcat_ref[...])

        for c in range(NC):
            rd["cw", 0, c].wait_recv()
            rd["cw", 1, c] = make_chunk(1, c, ccw=False)
            rd["ccw", 0, c].wait_recv()
            rd["ccw", 1, c] = make_chunk(1, c, ccw=True)
        acc = acc + compute_block(wcw_ref[0])
        acc = acc + compute_block(wccw_ref[0])

        for c in range(NC):
            rd["cw", 1, c].wait_recv()
            rd["cw", 2, c] = make_chunk(2, c, ccw=False)
            rd["ccw", 1, c].wait_recv()
            rd["ccw", 2, c] = make_chunk(2, c, ccw=True)
        acc = acc + compute_block(wcw_ref[1])
        acc = acc + compute_block(wccw_ref[1])

        rd["cw", 2, 0].wait_recv()
        ws_cw = attn_weights(wcw_ref[2, 0:2 * D])
        rd["ccw", 2, 0].wait_recv()
        ws_ccw = attn_weights(wccw_ref[2, 0:2 * D])
        rd["cw", 2, 1].wait_recv()
        attn_ctx(wcw_ref[2, 2 * D:3 * D], ws_cw)
        rd["cw", 2, 2].wait_recv()
        acc = acc + out_proj(wcw_ref[2, 3 * D:4 * D])
        rd["ccw", 2, 1].wait_recv()
        attn_ctx(wccw_ref[2, 2 * D:3 * D], ws_ccw)
        rd["ccw", 2, 2].wait_recv()
        acc = acc + out_proj(wccw_ref[2, 3 * D:4 * D])

        for key in rd:
            rd[key].wait_send()

        out_ref[...] = acc.reshape(Bl, Sq, D)

    return pl.pallas_call(
        body,
        out_shape=jax.ShapeDtypeStruct((Bl, Sq, D), jnp.float32),
        in_specs=[pl.BlockSpec(memory_space=pltpu.VMEM)] * 5,
        out_specs=pl.BlockSpec(memory_space=pltpu.VMEM),
        scratch_shapes=[
            pltpu.VMEM((NH, 4 * D, HW), jnp.bfloat16),
            pltpu.VMEM((NH, 4 * D, HW), jnp.bfloat16),
            pltpu.VMEM((BS, HD), jnp.bfloat16),
            pltpu.SemaphoreType.DMA((NH, 3)),
            pltpu.SemaphoreType.DMA((NH, 3)),
            pltpu.SemaphoreType.DMA((NH, 3)),
            pltpu.SemaphoreType.DMA((NH, 3)),
        ],
        compiler_params=pltpu.CompilerParams(collective_id=0),
    )(x.astype(jnp.bfloat16), wcat, cos_t, sin_t, P_m)


# device time: 31633 ns/iter; 1.2128x vs baseline; 1.2128x over previous
import numpy as np
import jax
import jax.numpy as jnp
from jax import lax
from jax.experimental import pallas as pl
from jax.experimental.pallas import tpu as pltpu

N_DEV = 4


def _rope_tables(Sq, Dh, n_heads, Bl):
    inv = 1.0 / (10000.0 ** (np.arange(0, Dh, 2) / Dh))
    pos = np.arange(Sq)[:, None] * inv[None, :]
    cos = np.repeat(np.cos(pos), 2, axis=-1).astype(np.float32)
    sin = np.repeat(np.sin(pos), 2, axis=-1).astype(np.float32)
    cos_t = np.tile(np.tile(cos, (1, n_heads)), (Bl, 1))
    sin_t = np.tile(np.tile(sin, (1, n_heads)), (Bl, 1))
    n = n_heads * Dh
    P = np.zeros((n, n), np.float32)
    for h in range(n_heads):
        o = h * Dh
        for k in range(Dh // 2):
            P[o + 2 * k + 1, o + 2 * k] = -1.0
            P[o + 2 * k, o + 2 * k + 1] = 1.0
    return cos_t, sin_t, P


def kernel(x, Wq, Wk, Wv, Wo):
    Bl, Sq, D = x.shape
    HD = Wq.shape[1]
    Dh = 64
    HW = HD // 2
    BS = Bl * Sq
    NH = N_DEV - 1

    wcat_f = jnp.concatenate([Wq, Wk, Wv, Wo.T], axis=0)
    wm = wcat_f.reshape(4, D, HD)
    scl = jnp.maximum(jnp.max(jnp.abs(wm), axis=1), 1e-8)
    w8 = jnp.round(wm / scl[:, None, :] * 127.0).astype(jnp.int8)
    w8 = w8.reshape(4 * D, HD)
    scl_over = jnp.zeros((8, HD), jnp.float32).at[0:4].set(scl / 127.0)

    cos_np, sin_np, P_np = _rope_tables(Sq, Dh, HD // Dh, Bl)
    cos_t = jnp.asarray(cos_np)
    sin_t = jnp.asarray(sin_np)
    P_m = jnp.asarray(P_np).astype(jnp.bfloat16)

    def body(x_ref, w8_ref, scl_ref, cos_ref, sin_ref, p_ref, out_ref,
             wcw_ref, wccw_ref, scw_ref, sccw_ref, ctx_ref,
             cw_send, cw_recv, ccw_send, ccw_recv):
        me = lax.axis_index("i")
        left = lax.rem(me + N_DEV - 1, N_DEV)
        right = lax.rem(me + 1, N_DEV)

        barrier = pltpu.get_barrier_semaphore()
        for nbr in (left, right):
            pl.semaphore_signal(
                barrier, inc=1,
                device_id=(nbr,), device_id_type=pl.DeviceIdType.MESH,
            )
        pl.semaphore_wait(barrier, 2)

        CHUNK_ROWS = ((0, 2 * D), (2 * D, 3 * D), (3 * D, 4 * D))
        SC = 3
        CHUNK_ORDER = (SC, 0, 1, 2)

        def make_chunk(u, c, ccw):
            sems = (ccw_send, ccw_recv) if ccw else (cw_send, cw_recv)
            tgt = left if ccw else right
            if c == SC:
                s_ref = sccw_ref if ccw else scw_ref
                if u == 0:
                    cols = (HW, HD) if ccw else (0, HW)
                    src = scl_ref.at[:, cols[0]:cols[1]]
                else:
                    src = s_ref.at[u - 1]
                dst = s_ref.at[u]
            else:
                w_ref = wccw_ref if ccw else wcw_ref
                lo, hi = CHUNK_ROWS[c]
                if u == 0:
                    cols = (HW, HD) if ccw else (0, HW)
                    src = w8_ref.at[lo:hi, cols[0]:cols[1]]
                else:
                    src = w_ref.at[u - 1, lo:hi, :]
                dst = w_ref.at[u, lo:hi, :]
            r = pltpu.make_async_remote_copy(
                src_ref=src, dst_ref=dst,
                send_sem=sems[0].at[u, c], recv_sem=sems[1].at[u, c],
                device_id=(tgt,), device_id_type=pl.DeviceIdType.MESH,
            )
            r.start()
            return r

        rd = {}
        for c in CHUNK_ORDER:
            rd["cw", 0, c] = make_chunk(0, c, ccw=False)
            rd["ccw", 0, c] = make_chunk(0, c, ccw=True)

        x2 = x_ref[...].reshape(BS, D)
        cos2 = cos_ref[...]
        sin2 = sin_ref[...]
        pm = p_ref[...]

        def attn_weights(wqk8, sv):
            width = wqk8.shape[1]
            cw_ = cos2[:, :width]
            sw_ = sin2[:, :width]
            pw_ = pm[:width, :width]
            q = jnp.dot(
                x2, wqk8[0:D].astype(jnp.bfloat16),
                preferred_element_type=jnp.float32,
            ) * sv[0:1, :]
            k = jnp.dot(
                x2, wqk8[D:2 * D].astype(jnp.bfloat16),
                preferred_element_type=jnp.float32,
            ) * sv[1:2, :]
            qp = jnp.dot(q.astype(jnp.bfloat16), pw_, preferred_element_type=jnp.float32)
            kp = jnp.dot(k.astype(jnp.bfloat16), pw_, preferred_element_type=jnp.float32)
            q = (q * cw_ + qp * sw_).astype(jnp.bfloat16)
            k = (k * cw_ + kp * sw_).astype(jnp.bfloat16)
            ws = {}
            for b in range(Bl):
                for h in range(width // Dh):
                    qh = q[b * Sq:(b + 1) * Sq, h * Dh:(h + 1) * Dh]
                    kh = k[b * Sq:(b + 1) * Sq, h * Dh:(h + 1) * Dh]
                    s = lax.dot_general(
                        qh, kh, (((1,), (1,)), ((), ())),
                        preferred_element_type=jnp.float32,
                    ) * 0.125
                    w = jnp.exp(s)
                    denom = jnp.sum(w, axis=-1, keepdims=True)
                    ws[b, h] = (w.astype(jnp.bfloat16), denom)
            return ws

        def attn_ctx(wv8, sv, ws):
            width = wv8.shape[1]
            v = (jnp.dot(
                x2, wv8.astype(jnp.bfloat16),
                preferred_element_type=jnp.float32,
            ) * sv[2:3, :]).astype(jnp.bfloat16)
            for b in range(Bl):
                for h in range(width // Dh):
                    vh = v[b * Sq:(b + 1) * Sq, h * Dh:(h + 1) * Dh]
                    w, denom = ws[b, h]
                    cx = jnp.dot(
                        w, vh, preferred_element_type=jnp.float32
                    ) / denom
                    ctx_ref[b * Sq:(b + 1) * Sq, h * Dh:(h + 1) * Dh] = (
                        cx.astype(jnp.bfloat16)
                    )

        def out_proj(woT8, sv):
            width = woT8.shape[1]
            cs = (ctx_ref[:, :width] * sv[3:4, :]).astype(jnp.bfloat16)
            return lax.dot_general(
                cs, woT8.astype(jnp.bfloat16), (((1,), (1,)), ((), ())),
                preferred_element_type=jnp.float32,
            )

        def compute_block(wblk8, sv):
            ws = attn_weights(wblk8[0:2 * D], sv)
            attn_ctx(wblk8[2 * D:3 * D], sv, ws)
            return out_proj(wblk8[3 * D:4 * D], sv)

        acc = compute_block(w8_ref[...], scl_ref[...])

        for c in CHUNK_ORDER:
            rd["cw", 0, c].wait_recv()
            rd["cw", 1, c] = make_chunk(1, c, ccw=False)
            rd["ccw", 0, c].wait_recv()
            rd["ccw", 1, c] = make_chunk(1, c, ccw=True)
        acc = acc + compute_block(wcw_ref[0], scw_ref[0])
        acc = acc + compute_block(wccw_ref[0], sccw_ref[0])

        for c in CHUNK_ORDER:
            rd["cw", 1, c].wait_recv()
            rd["cw", 2, c] = make_chunk(2, c, ccw=False)
            rd["ccw", 1, c].wait_recv()
            rd["ccw", 2, c] = make_chunk(2, c, ccw=True)
        acc = acc + compute_block(wcw_ref[1], scw_ref[1])
        acc = acc + compute_block(wccw_ref[1], sccw_ref[1])

        rd["cw", 2, SC].wait_recv()
        rd["ccw", 2, SC].wait_recv()
        rd["cw", 2, 0].wait_recv()
        ws_cw = attn_weights(wcw_ref[2, 0:2 * D], scw_ref[2])
        rd["ccw", 2, 0].wait_recv()
        ws_ccw = attn_weights(wccw_ref[2, 0:2 * D], sccw_ref[2])
        rd["cw", 2, 1].wait_recv()
        attn_ctx(wcw_ref[2, 2 * D:3 * D], scw_ref[2], ws_cw)
        rd["cw", 2, 2].wait_recv()
        acc = acc + out_proj(wcw_ref[2, 3 * D:4 * D], scw_ref[2])
        rd["ccw", 2, 1].wait_recv()
        attn_ctx(wccw_ref[2, 2 * D:3 * D], sccw_ref[2], ws_ccw)
        rd["ccw", 2, 2].wait_recv()
        acc = acc + out_proj(wccw_ref[2, 3 * D:4 * D], sccw_ref[2])

        for key in rd:
            rd[key].wait_send()

        out_ref[...] = acc.reshape(Bl, Sq, D)

    return pl.pallas_call(
        body,
        out_shape=jax.ShapeDtypeStruct((Bl, Sq, D), jnp.float32),
        in_specs=[pl.BlockSpec(memory_space=pltpu.VMEM)] * 6,
        out_specs=pl.BlockSpec(memory_space=pltpu.VMEM),
        scratch_shapes=[
            pltpu.VMEM((NH, 4 * D, HW), jnp.int8),
            pltpu.VMEM((NH, 4 * D, HW), jnp.int8),
            pltpu.VMEM((NH, 8, HW), jnp.float32),
            pltpu.VMEM((NH, 8, HW), jnp.float32),
            pltpu.VMEM((BS, HD), jnp.bfloat16),
            pltpu.SemaphoreType.DMA((NH, 4)),
            pltpu.SemaphoreType.DMA((NH, 4)),
            pltpu.SemaphoreType.DMA((NH, 4)),
            pltpu.SemaphoreType.DMA((NH, 4)),
        ],
        compiler_params=pltpu.CompilerParams(collective_id=0),
    )(x.astype(jnp.bfloat16), w8, scl_over, cos_t, sin_t, P_m)


# device time: 29995 ns/iter; 1.2791x vs baseline; 1.0546x over previous
import numpy as np
import jax
import jax.numpy as jnp
from jax import lax
from jax.experimental import pallas as pl
from jax.experimental.pallas import tpu as pltpu

N_DEV = 4


def _rope_tables(Sq, Dh, n_heads, Bl):
    inv = 1.0 / (10000.0 ** (np.arange(0, Dh, 2) / Dh))
    pos = np.arange(Sq)[:, None] * inv[None, :]
    cos = np.repeat(np.cos(pos), 2, axis=-1).astype(np.float32)
    sin = np.repeat(np.sin(pos), 2, axis=-1).astype(np.float32)
    cos_t = np.tile(np.tile(cos, (1, n_heads)), (Bl, 1))
    sin_t = np.tile(np.tile(sin, (1, n_heads)), (Bl, 1))
    n = n_heads * Dh
    P = np.zeros((n, n), np.float32)
    for h in range(n_heads):
        o = h * Dh
        for k in range(Dh // 2):
            P[o + 2 * k + 1, o + 2 * k] = -1.0
            P[o + 2 * k, o + 2 * k + 1] = 1.0
    return cos_t, sin_t, P


def _quant_cols(w):
    s = jnp.maximum(jnp.max(jnp.abs(w), axis=0, keepdims=True), 1e-8)
    return jnp.round(w / s * 127.0).astype(jnp.int8), s / 127.0


def kernel(x, Wq, Wk, Wv, Wo):
    Bl, Sq, D = x.shape
    HD = Wq.shape[1]
    Dh = 64
    HW = HD // 2
    BS = Bl * Sq
    NH = N_DEV - 1

    wq8, sq = _quant_cols(Wq)
    wk8, sk = _quant_cols(Wk)
    wv8, sv_ = _quant_cols(Wv)
    wo8, so = _quant_cols(Wo.T)
    scl = jnp.zeros((8, HD), jnp.float32)
    scl = scl.at[0:1].set(sq).at[1:2].set(sk).at[2:3].set(sv_).at[3:4].set(so)
    wo8 = wo8.T

    cos_np, sin_np, P_np = _rope_tables(Sq, Dh, HD // Dh, Bl)
    cos_t = jnp.asarray(cos_np)
    sin_t = jnp.asarray(sin_np)
    P_m = jnp.asarray(P_np).astype(jnp.bfloat16)

    def body(x_ref, wq_ref, wk_ref, wv_ref, wo_ref, scl_ref,
             cos_ref, sin_ref, p_ref, out_ref,
             qcw_ref, kcw_ref, vcw_ref, ocw_ref, scw_ref,
             qccw_ref, kccw_ref, vccw_ref, occw_ref, sccw_ref,
             ctx_ref, cw_send, cw_recv, ccw_send, ccw_recv):
        me = lax.axis_index("i")
        left = lax.rem(me + N_DEV - 1, N_DEV)
        right = lax.rem(me + 1, N_DEV)

        barrier = pltpu.get_barrier_semaphore()
        for nbr in (left, right):
            pl.semaphore_signal(
                barrier, inc=1,
                device_id=(nbr,), device_id_type=pl.DeviceIdType.MESH,
            )
        pl.semaphore_wait(barrier, 2)

        CW = ((scw_ref, scl_ref, 1), (qcw_ref, wq_ref, 1),
              (kcw_ref, wk_ref, 1), (vcw_ref, wv_ref, 1),
              (ocw_ref, wo_ref, 0))
        CCW = ((sccw_ref, scl_ref, 1), (qccw_ref, wq_ref, 1),
               (kccw_ref, wk_ref, 1), (vccw_ref, wv_ref, 1),
               (occw_ref, wo_ref, 0))
        NCH = len(CW)

        def make_chunk(u, c, ccw):
            buf, inp, axis = (CCW if ccw else CW)[c]
            if u == 0:
                second = ccw
                if axis == 1:
                    src = inp.at[:, HW:HD] if second else inp.at[:, 0:HW]
                else:
                    src = inp.at[HW:HD, :] if second else inp.at[0:HW, :]
            else:
                src = buf.at[u - 1]
            sems = (ccw_send, ccw_recv) if ccw else (cw_send, cw_recv)
            tgt = left if ccw else right
            r = pltpu.make_async_remote_copy(
                src_ref=src, dst_ref=buf.at[u],
                send_sem=sems[0].at[u, c], recv_sem=sems[1].at[u, c],
                device_id=(tgt,), device_id_type=pl.DeviceIdType.MESH,
            )
            r.start()
            return r

        rd = {}
        for c in range(NCH):
            rd["cw", 0, c] = make_chunk(0, c, ccw=False)
            rd["ccw", 0, c] = make_chunk(0, c, ccw=True)

        x2 = x_ref[...].reshape(BS, D)
        cos2 = cos_ref[...]
        sin2 = sin_ref[...]
        pm = p_ref[...]

        def attn_weights(wq8_, wk8_, sv):
            width = wq8_.shape[1]
            cw_ = cos2[:, :width]
            sw_ = sin2[:, :width]
            pw_ = pm[:width, :width]
            q = jnp.dot(
                x2, wq8_.astype(jnp.bfloat16),
                preferred_element_type=jnp.float32,
            ) * sv[0:1, :]
            k = jnp.dot(
                x2, wk8_.astype(jnp.bfloat16),
                preferred_element_type=jnp.float32,
            ) * sv[1:2, :]
            qp = jnp.dot(q.astype(jnp.bfloat16), pw_, preferred_element_type=jnp.float32)
            kp = jnp.dot(k.astype(jnp.bfloat16), pw_, preferred_element_type=jnp.float32)
            q = (q * cw_ + qp * sw_).astype(jnp.bfloat16)
            k = (k * cw_ + kp * sw_).astype(jnp.bfloat16)
            ws = {}
            for b in range(Bl):
                for h in range(width // Dh):
                    qh = q[b * Sq:(b + 1) * Sq, h * Dh:(h + 1) * Dh]
                    kh = k[b * Sq:(b + 1) * Sq, h * Dh:(h + 1) * Dh]
                    s = lax.dot_general(
                        qh, kh, (((1,), (1,)), ((), ())),
                        preferred_element_type=jnp.float32,
                    ) * 0.125
                    w = jnp.exp(s)
                    denom = jnp.sum(w, axis=-1, keepdims=True)
                    ws[b, h] = (w.astype(jnp.bfloat16), denom)
            return ws

        def attn_ctx(wv8_, sv, ws):
            width = wv8_.shape[1]
            v = (jnp.dot(
                x2, wv8_.astype(jnp.bfloat16),
                preferred_element_type=jnp.float32,
            ) * sv[2:3, :]).astype(jnp.bfloat16)
            for b in range(Bl):
                for h in range(width // Dh):
                    vh = v[b * Sq:(b + 1) * Sq, h * Dh:(h + 1) * Dh]
                    w, denom = ws[b, h]
                    cx = jnp.dot(
                        w, vh, preferred_element_type=jnp.float32
                    ) / denom
                    ctx_ref[b * Sq:(b + 1) * Sq, h * Dh:(h + 1) * Dh] = (
                        cx.astype(jnp.bfloat16)
                    )

        def out_proj(wo8_, sv):
            width = wo8_.shape[0]
            cs = (ctx_ref[:, :width] * sv[3:4, :width]).astype(jnp.bfloat16)
            return lax.dot_general(
                cs, wo8_.astype(jnp.bfloat16), (((1,), (0,)), ((), ())),
                preferred_element_type=jnp.float32,
            )

        def compute_block(wq8_, wk8_, wv8_, wo8_, sv):
            ws = attn_weights(wq8_, wk8_, sv)
            attn_ctx(wv8_, sv, ws)
            return out_proj(wo8_, sv)

        acc = compute_block(
            wq_ref[...], wk_ref[...], wv_ref[...], wo_ref[...], scl_ref[...]
        )

        def hop_block(side, u):
            bufs = CCW if side == "ccw" else CW
            return compute_block(
                bufs[1][0][u], bufs[2][0][u], bufs[3][0][u], bufs[4][0][u],
                bufs[0][0][u],
            )

        for u in (0, 1):
            for c in range(NCH):
                rd["cw", u, c].wait_recv()
                rd["cw", u + 1, c] = make_chunk(u + 1, c, ccw=False)
                rd["ccw", u, c].wait_recv()
                rd["ccw", u + 1, c] = make_chunk(u + 1, c, ccw=True)
            acc = acc + hop_block("cw", u)
            acc = acc + hop_block("ccw", u)

        for c in (0, 1, 2):
            rd["cw", 2, c].wait_recv()
            rd["ccw", 2, c].wait_recv()
        ws_cw = attn_weights(qcw_ref[2], kcw_ref[2], scw_ref[2])
        ws_ccw = attn_weights(qccw_ref[2], kccw_ref[2], sccw_ref[2])
        rd["cw", 2, 3].wait_recv()
        attn_ctx(vcw_ref[2], scw_ref[2], ws_cw)
        rd["cw", 2, 4].wait_recv()
        acc = acc + out_proj(ocw_ref[2], scw_ref[2])
        rd["ccw", 2, 3].wait_recv()
        attn_ctx(vccw_ref[2], sccw_ref[2], ws_ccw)
        rd["ccw", 2, 4].wait_recv()
        acc = acc + out_proj(occw_ref[2], sccw_ref[2])

        for key in rd:
            rd[key].wait_send()

        out_ref[...] = acc.reshape(Bl, Sq, D)

    return pl.pallas_call(
        body,
        out_shape=jax.ShapeDtypeStruct((Bl, Sq, D), jnp.float32),
        in_specs=[pl.BlockSpec(memory_space=pltpu.VMEM)] * 9,
        out_specs=pl.BlockSpec(memory_space=pltpu.VMEM),
        scratch_shapes=[
            pltpu.VMEM((NH, D, HW), jnp.int8),
            pltpu.VMEM((NH, D, HW), jnp.int8),
            pltpu.VMEM((NH, D, HW), jnp.int8),
            pltpu.VMEM((NH, HW, D), jnp.int8),
            pltpu.VMEM((NH, 8, HW), jnp.float32),
            pltpu.VMEM((NH, D, HW), jnp.int8),
            pltpu.VMEM((NH, D, HW), jnp.int8),
            pltpu.VMEM((NH, D, HW), jnp.int8),
            pltpu.VMEM((NH, HW, D), jnp.int8),
            pltpu.VMEM((NH, 8, HW), jnp.float32),
            pltpu.VMEM((BS, HD), jnp.bfloat16),
            pltpu.SemaphoreType.DMA((NH, 5)),
            pltpu.SemaphoreType.DMA((NH, 5)),
            pltpu.SemaphoreType.DMA((NH, 5)),
            pltpu.SemaphoreType.DMA((NH, 5)),
        ],
        compiler_params=pltpu.CompilerParams(collective_id=0),
    )(x.astype(jnp.bfloat16), wq8, wk8, wv8, wo8, scl, cos_t, sin_t, P_m)
